# baseline (speedup 1.0000x reference)
import functools

import jax
import jax.numpy as jnp
from jax import lax
from jax.experimental import pallas as pl
from jax.experimental.pallas import tpu as pltpu
from jax.experimental.pallas import tpu_sc as plsc

_BATCH = 16384
_FIELDS = 100
_N = _BATCH * _FIELDS
_LANES = 16
_NUM_WORKERS = 32
_PER_W = _N // _NUM_WORKERS
_CHUNKS = 8
_C = _PER_W // _CHUNKS
_UNROLL = 8
_STEPS = _C // (_LANES * _UNROLL)

_mesh = plsc.VectorSubcoreMesh(core_axis_name="c", subcore_axis_name="s")


@functools.partial(
    pl.kernel,
    mesh=_mesh,
    out_type=jax.ShapeDtypeStruct((_NUM_WORKERS, _LANES), jnp.float32),
    scratch_types=[
        pltpu.VMEM((_C,), jnp.int32),
        pltpu.VMEM((_C,), jnp.int32),
        pltpu.VMEM((_C,), jnp.float32),
        pltpu.VMEM((_C,), jnp.float32),
        pltpu.VMEM((_LANES,), jnp.float32),
        pltpu.SemaphoreType.DMA,
        pltpu.SemaphoreType.DMA,
        pltpu.SemaphoreType.DMA,
        pltpu.SemaphoreType.DMA,
    ],
)
def _gather_sum(idx_hbm, w_hbm, out_hbm, idx_v0, idx_v1, vals_v0, vals_v1,
                acc_v, isem0, isem1, gsem0, gsem1):
    nc = plsc.get_sparse_core_info().num_cores
    wid = lax.axis_index("s") * nc + lax.axis_index("c")
    base = wid * _PER_W
    idx_bufs = (idx_v0, idx_v1)
    vals_bufs = (vals_v0, vals_v1)
    isems = (isem0, isem1)
    gsems = (gsem0, gsem1)

    def start_idx(c):
        s = c % 2
        return pltpu.async_copy(
            idx_hbm.at[pl.ds(base + c * _C, _C)], idx_bufs[s], isems[s])

    def start_gather(c):
        s = c % 2
        return pltpu.async_copy(w_hbm.at[idx_bufs[s]], vals_bufs[s], gsems[s])

    def reduce_chunk(c, accs):
        vals_v = vals_bufs[c % 2]

        def body(i, a):
            o = i * (_LANES * _UNROLL)
            return tuple(
                a[j] + vals_v[pl.ds(o + j * _LANES, _LANES)]
                for j in range(_UNROLL)
            )

        return lax.fori_loop(0, _STEPS, body, accs)

    h_idx = [None] * _CHUNKS
    h_g = [None] * _CHUNKS
    h_idx[0] = start_idx(0)
    h_idx[1] = start_idx(1)
    h_idx[0].wait()
    h_g[0] = start_gather(0)
    zeros = jnp.zeros((_LANES,), jnp.float32)
    accs = (zeros,) * _UNROLL
    for c in range(1, _CHUNKS):
        h_g[c - 1].wait()
        if c + 1 < _CHUNKS:
            h_idx[c + 1] = start_idx(c + 1)
        h_idx[c].wait()
        h_g[c] = start_gather(c)
        accs = reduce_chunk(c - 1, accs)
    h_g[_CHUNKS - 1].wait()
    accs = reduce_chunk(_CHUNKS - 1, accs)

    total = accs[0]
    for j in range(1, _UNROLL):
        total = total + accs[j]
    acc_v[...] = total
    pltpu.sync_copy(acc_v, out_hbm.at[wid])


def kernel(non_zero_indices, weights):
    idx = non_zero_indices.reshape(-1).astype(jnp.int32)
    partials = _gather_sum(idx, weights)
    return jnp.sum(partials)


# speedup vs baseline: 1.1748x; 1.1748x over previous
import functools

import jax
import jax.numpy as jnp
from jax import lax
from jax.experimental import pallas as pl
from jax.experimental.pallas import tpu as pltpu
from jax.experimental.pallas import tpu_sc as plsc

_BATCH = 16384
_FIELDS = 100
_LANES = 16
_NUM_WORKERS = 32
_ROWS_W = _BATCH // _NUM_WORKERS
_FULL = _FIELDS // _LANES
_TAIL_OFF = _FIELDS - _LANES
_TAIL_DUP = _LANES - (_FIELDS - _FULL * _LANES)

_mesh = plsc.VectorSubcoreMesh(core_axis_name="c", subcore_axis_name="s")


@functools.partial(
    pl.kernel,
    mesh=_mesh,
    out_type=jax.ShapeDtypeStruct((_NUM_WORKERS, _LANES), jnp.float32),
    scratch_types=[
        pltpu.VMEM((_ROWS_W // 2, _FIELDS), jnp.int32),
        pltpu.VMEM((_ROWS_W // 2, _FIELDS), jnp.float32),
        pltpu.VMEM((_LANES,), jnp.float32),
        pltpu.SemaphoreType.DMA,
    ],
)
def _gather_sum(idx_hbm, w_hbm, out_hbm, idx_v, vals_v, acc_v, sem):
    nc = plsc.get_sparse_core_info().num_cores
    wid = lax.axis_index("s") * nc + lax.axis_index("c")
    half = _ROWS_W // 2

    tail_mask = lax.iota(jnp.int32, _LANES) < _TAIL_DUP
    fzero = jnp.zeros((_LANES,), jnp.float32)
    accs = (fzero,) * (_FULL + 1)

    for h in range(2):
        pltpu.sync_copy(
            idx_hbm.at[pl.ds(wid * _ROWS_W + h * half, half), :], idx_v)

        def issue(r, carry):
            pltpu.async_copy(w_hbm.at[idx_v.at[r]], vals_v.at[r], sem)
            return carry

        lax.fori_loop(0, half, issue, 0)

        def drain(r, carry):
            pltpu.make_async_copy(
                w_hbm.at[idx_v.at[r]], vals_v.at[r], sem).wait()
            return carry

        lax.fori_loop(0, half, drain, 0)

        def body(r, a):
            new = [a[j] + vals_v[r, pl.ds(j * _LANES, _LANES)]
                   for j in range(_FULL)]
            tail = vals_v[r, pl.ds(_TAIL_OFF, _LANES)]
            new.append(a[_FULL] + jnp.where(tail_mask, fzero, tail))
            return tuple(new)

        accs = lax.fori_loop(0, half, body, accs)
    total = accs[0]
    for j in range(1, _FULL + 1):
        total = total + accs[j]
    acc_v[...] = total
    pltpu.sync_copy(acc_v, out_hbm.at[wid])


def kernel(non_zero_indices, weights):
    partials = _gather_sum(non_zero_indices, weights)
    return jnp.sum(partials)
